# Initial kernel scaffold; baseline (speedup 1.0000x reference)
import jax
import jax.numpy as jnp
from jax.experimental import pallas as pl


def kernel(x, edge_index, W1, b1, W2, b2):
    raise NotImplementedError("write your pallas kernel here")


# speedup vs baseline: 14.0935x; 14.0935x over previous
import functools

import jax
import jax.numpy as jnp
from jax import lax
from jax.experimental import pallas as pl
from jax.experimental.pallas import tpu as pltpu
from jax.experimental.pallas import tpu_sc as plsc

N_NODES = 10000
N_EDGES = 160000
PAD_N = 10240
NC = 2
NS = 16
ROWS_PER_TILE = PAD_N // NS

F32 = jnp.float32


BD = 40
KD = N_EDGES // (NC * NS * BD)


def _deg_body(dst_hbm, ones_hbm, zeros_hbm, out, didx_v, ones_v, acc_sh):
    c = lax.axis_index("c")
    s = lax.axis_index("s")
    pltpu.sync_copy(zeros_hbm.at[pl.ds(s * ROWS_PER_TILE, ROWS_PER_TILE)],
                    acc_sh.at[pl.ds(s * ROWS_PER_TILE, ROWS_PER_TILE)])
    pltpu.sync_copy(ones_hbm, ones_v)
    pltpu.sync_copy(dst_hbm.at[c].at[s], didx_v)
    plsc.subcore_barrier()

    def step(j, carry):
        pltpu.sync_copy(ones_v, acc_sh.at[didx_v.at[j]], add=True)
        return carry

    lax.fori_loop(0, KD, step, 0)
    plsc.subcore_barrier()
    pltpu.sync_copy(acc_sh.at[pl.ds(s * ROWS_PER_TILE, ROWS_PER_TILE)],
                    out.at[c].at[pl.ds(s * ROWS_PER_TILE, ROWS_PER_TILE)])


def _run_deg(dst):
    mesh = plsc.VectorSubcoreMesh(core_axis_name="c", subcore_axis_name="s",
                                  num_cores=NC, num_subcores=NS)
    dst_r = dst.reshape(NC, NS, KD, BD)
    ones = jnp.ones((BD, DW), F32)
    zeros = jnp.zeros((PAD_N, DW), F32)
    k = pl.kernel(
        _deg_body,
        out_type=[jax.ShapeDtypeStruct((NC, PAD_N, DW), F32)],
        mesh=mesh,
        scratch_types=[
            pltpu.VMEM((KD, BD), jnp.int32),
            pltpu.VMEM((BD, DW), F32),
            pltpu.VMEM_SHARED((PAD_N, DW), F32),
        ],
    )
    return k(dst_r, ones, zeros)[0]


DW = 128


def _scatter_body(n_tables, n_outs, k_b, tasks, *refs):
    k, b = k_b
    (src_hbm, dst_hbm, zeros_hbm), refs = refs[:3], refs[3:]
    tabs, refs = refs[:n_tables], refs[n_tables:]
    outs, refs = refs[:n_outs], refs[n_outs:]
    sidx_v, didx_v, rows_a, rows_b, acc_sh, gsem = refs
    c = lax.axis_index("c")
    s = lax.axis_index("s")

    for core_id in range(NC):
        for ti, oi, ei in tasks[core_id]:
            @pl.when(c == core_id)
            def _(ti=ti, oi=oi, ei=ei):
                pltpu.sync_copy(src_hbm.at[ei, s], sidx_v)
                pltpu.sync_copy(dst_hbm.at[ei, s], didx_v)
                pltpu.sync_copy(
                    zeros_hbm.at[pl.ds(s * ROWS_PER_TILE, ROWS_PER_TILE)],
                    acc_sh.at[pl.ds(s * ROWS_PER_TILE, ROWS_PER_TILE)])
                plsc.subcore_barrier()

                pltpu.async_copy(tabs[ti].at[sidx_v.at[pl.ds(0, b)]],
                                 rows_a, gsem).wait()

                def pair(i, carry):
                    j = 2 * i
                    cp = pltpu.async_copy(
                        tabs[ti].at[sidx_v.at[pl.ds((j + 1) * b, b)]],
                        rows_b, gsem)
                    pltpu.sync_copy(rows_a, acc_sh.at[didx_v.at[j]], add=True)
                    cp.wait()
                    cp2 = pltpu.async_copy(
                        tabs[ti].at[sidx_v.at[pl.ds((j + 2) * b, b)]],
                        rows_a, gsem)
                    pltpu.sync_copy(rows_b, acc_sh.at[didx_v.at[j + 1]],
                                    add=True)
                    cp2.wait()
                    return carry

                lax.fori_loop(0, (k - 1) // 2, pair, 0)
                pltpu.sync_copy(rows_a, acc_sh.at[didx_v.at[k - 1]], add=True)
                plsc.subcore_barrier()
                pltpu.sync_copy(
                    acc_sh.at[pl.ds(s * ROWS_PER_TILE, ROWS_PER_TILE)],
                    outs[oi].at[pl.ds(s * ROWS_PER_TILE, ROWS_PER_TILE)])


def _run_scatter(src, dst, tables, n_outs, n_eslices, k, b, tasks):
    n_tables = len(tables)
    mesh = plsc.VectorSubcoreMesh(core_axis_name="c", subcore_axis_name="s",
                                  num_cores=NC, num_subcores=NS)
    src_r = src.reshape(n_eslices, NS, k * b)
    dst_r = dst.reshape(n_eslices, NS, k, b)
    zeros = jnp.zeros((PAD_N, DW), F32)
    out_ty = [jax.ShapeDtypeStruct((PAD_N, DW), F32)] * n_outs
    body = functools.partial(_scatter_body, n_tables, n_outs, (k, b), tasks)
    kern = pl.kernel(
        body,
        out_type=out_ty,
        mesh=mesh,
        scratch_types=[
            pltpu.VMEM((k * b,), jnp.int32),
            pltpu.VMEM((k, b), jnp.int32),
            pltpu.VMEM((b, DW), F32),
            pltpu.VMEM((b, DW), F32),
            pltpu.VMEM_SHARED((PAD_N, DW), F32),
            pltpu.SemaphoreType.DMA,
        ],
    )
    return kern(src_r, dst_r, zeros, *tables)


def _tc_prep_body(deg3, x, dinv_o, y1_o):
    deg = deg3[0, :, 0:1] + deg3[1, :, 0:1] + 1.0
    dinv = lax.rsqrt(deg)
    dinv_o[...] = dinv
    y1_o[...] = x[...] * dinv


def _run_prep(deg3, x_pad):
    return pl.pallas_call(
        _tc_prep_body,
        out_shape=[
            jax.ShapeDtypeStruct((PAD_N, 1), F32),
            jax.ShapeDtypeStruct((PAD_N, 128), F32),
        ],
    )(deg3, x_pad)


TN = 640


def _tc_mid_body(dinv, s1a, s1b, y1, w1, b1, w2, *outs):
    di = dinv[...]
    agg = (s1a[...] + s1b[...] + y1[...]) * di
    h = jnp.maximum(jnp.dot(agg, w1[...], preferred_element_type=F32) + b1[...],
                    0.0)
    t = jnp.dot(h, w2[...], preferred_element_type=F32) * di
    for j, o in enumerate(outs):
        o[...] = t[:, 128 * j:128 * (j + 1)]


def _run_mid(dinv, s1a, s1b, y1, w1, b1, w2p):
    grid = (PAD_N // TN,)
    node_spec = pl.BlockSpec((TN, 128), lambda i: (i, 0))
    return pl.pallas_call(
        _tc_mid_body,
        grid=grid,
        in_specs=[
            pl.BlockSpec((TN, 1), lambda i: (i, 0)),
            node_spec, node_spec, node_spec,
            pl.BlockSpec((128, 1600), lambda i: (0, 0)),
            pl.BlockSpec((1, 1600), lambda i: (0, 0)),
            pl.BlockSpec((1600, 512), lambda i: (0, 0)),
        ],
        out_specs=[pl.BlockSpec((TN, 128), lambda i: (i, 0))] * 4,
        out_shape=[jax.ShapeDtypeStruct((PAD_N, 128), F32)] * 4,
    )(dinv, s1a, s1b, y1, w1, b1, w2p)


def _tc_final_body(*refs):
    dinv, b2 = refs[0], refs[1]
    s2 = refs[2:6]
    y2 = refs[6:10]
    out = refs[10]
    t = jnp.concatenate([s2[j][...] + y2[j][...] for j in range(4)], axis=1)
    out[...] = t[:, :400] * dinv[...] + b2[...]


def _run_final(dinv, s2, y2, b2):
    grid = (PAD_N // TN,)
    chunk_spec = pl.BlockSpec((TN, 128), lambda i: (i, 0))
    return pl.pallas_call(
        _tc_final_body,
        grid=grid,
        in_specs=[pl.BlockSpec((TN, 1), lambda i: (i, 0)),
                  pl.BlockSpec((1, 400), lambda i: (0, 0))]
                 + [chunk_spec] * 8,
        out_specs=pl.BlockSpec((TN, 400), lambda i: (i, 0)),
        out_shape=jax.ShapeDtypeStruct((PAD_N, 400), F32),
    )(dinv, b2, *s2, *y2)


@jax.jit
def _kernel_impl(x, edge_index, W1, b1, W2, b2):
    src = edge_index[0]
    dst = edge_index[1]
    x_pad = jnp.zeros((PAD_N, 128), F32).at[:N_NODES].set(x)
    w2p = jnp.pad(W2, ((0, 0), (0, 112)))

    deg3 = _run_deg(dst)
    dinv, y1 = _run_prep(deg3, x_pad)
    s1a, s1b = _run_scatter(src, dst, [y1], 2, 2, 125, 40,
                            (((0, 0, 0),), ((0, 1, 1),)))
    y2 = _run_mid(dinv, s1a, s1b, y1, W1, b1.reshape(1, 1600), w2p)
    s2 = _run_scatter(src, dst, list(y2), 4, 1, 125, 80,
                      (((0, 0, 0), (2, 2, 0)), ((1, 1, 0), (3, 3, 0))))
    z_pad = _run_final(dinv, list(s2), list(y2), b2.reshape(1, 400))
    return z_pad[:N_NODES]


def kernel(x, edge_index, W1, b1, W2, b2):
    return _kernel_impl(x, edge_index, W1, b1, W2, b2)


# speedup vs baseline: 15.9508x; 1.1318x over previous
import functools

import jax
import jax.numpy as jnp
from jax import lax
from jax.experimental import pallas as pl
from jax.experimental.pallas import tpu as pltpu
from jax.experimental.pallas import tpu_sc as plsc

N_NODES = 10000
N_EDGES = 160000
PAD_N = 10240
NC = 2
NS = 16
ROWS_PER_TILE = PAD_N // NS

F32 = jnp.float32


BD = 40
KD = N_EDGES // (NC * NS * BD)


def _deg_body(dst_hbm, ones_hbm, zeros_hbm, out, didx_v, ones_v, acc_sh):
    c = lax.axis_index("c")
    s = lax.axis_index("s")
    pltpu.sync_copy(zeros_hbm.at[pl.ds(s * ROWS_PER_TILE, ROWS_PER_TILE)],
                    acc_sh.at[pl.ds(s * ROWS_PER_TILE, ROWS_PER_TILE)])
    pltpu.sync_copy(ones_hbm, ones_v)
    pltpu.sync_copy(dst_hbm.at[c].at[s], didx_v)
    plsc.subcore_barrier()

    def step(j, carry):
        pltpu.sync_copy(ones_v, acc_sh.at[didx_v.at[j]], add=True)
        return carry

    lax.fori_loop(0, KD, step, 0)
    plsc.subcore_barrier()
    pltpu.sync_copy(acc_sh.at[pl.ds(s * ROWS_PER_TILE, ROWS_PER_TILE)],
                    out.at[c].at[pl.ds(s * ROWS_PER_TILE, ROWS_PER_TILE)])


def _run_deg(dst):
    mesh = plsc.VectorSubcoreMesh(core_axis_name="c", subcore_axis_name="s",
                                  num_cores=NC, num_subcores=NS)
    dst_r = dst.reshape(NC, NS, KD, BD)
    ones = jnp.ones((BD, DW), F32)
    zeros = jnp.zeros((PAD_N, DW), F32)
    k = pl.kernel(
        _deg_body,
        out_type=[jax.ShapeDtypeStruct((NC, PAD_N, DW), F32)],
        mesh=mesh,
        scratch_types=[
            pltpu.VMEM((KD, BD), jnp.int32),
            pltpu.VMEM((BD, DW), F32),
            pltpu.VMEM_SHARED((PAD_N, DW), F32),
        ],
    )
    return k(dst_r, ones, zeros)[0]


DW = 128


def _scatter_body(n_tables, n_outs, k_b, tasks, *refs):
    k, b = k_b
    (src_hbm, dst_hbm, zeros_hbm), refs = refs[:3], refs[3:]
    tabs, refs = refs[:n_tables], refs[n_tables:]
    outs, refs = refs[:n_outs], refs[n_outs:]
    sidx_v, didx_v, rows_a, rows_b, acc_sh, gsem = refs
    c = lax.axis_index("c")
    s = lax.axis_index("s")

    for core_id in range(NC):
        for ti, oi, ei in tasks[core_id]:
            @pl.when(c == core_id)
            def _(ti=ti, oi=oi, ei=ei):
                pltpu.sync_copy(src_hbm.at[ei, s], sidx_v)
                pltpu.sync_copy(dst_hbm.at[ei, s], didx_v)
                pltpu.sync_copy(
                    zeros_hbm.at[pl.ds(s * ROWS_PER_TILE, ROWS_PER_TILE)],
                    acc_sh.at[pl.ds(s * ROWS_PER_TILE, ROWS_PER_TILE)])
                plsc.subcore_barrier()

                pltpu.async_copy(tabs[ti].at[sidx_v.at[pl.ds(0, b)]],
                                 rows_a, gsem).wait()

                def pair(i, carry):
                    j = 2 * i
                    cp = pltpu.async_copy(
                        tabs[ti].at[sidx_v.at[pl.ds((j + 1) * b, b)]],
                        rows_b, gsem)
                    pltpu.sync_copy(rows_a, acc_sh.at[didx_v.at[j]], add=True)
                    cp.wait()
                    cp2 = pltpu.async_copy(
                        tabs[ti].at[sidx_v.at[pl.ds((j + 2) * b, b)]],
                        rows_a, gsem)
                    pltpu.sync_copy(rows_b, acc_sh.at[didx_v.at[j + 1]],
                                    add=True)
                    cp2.wait()
                    return carry

                lax.fori_loop(0, (k - 1) // 2, pair, 0)
                pltpu.sync_copy(rows_a, acc_sh.at[didx_v.at[k - 1]], add=True)
                plsc.subcore_barrier()
                pltpu.sync_copy(
                    acc_sh.at[pl.ds(s * ROWS_PER_TILE, ROWS_PER_TILE)],
                    outs[oi].at[pl.ds(s * ROWS_PER_TILE, ROWS_PER_TILE)])


def _run_scatter(src, dst, tables, n_outs, n_eslices, k, b, tasks):
    n_tables = len(tables)
    mesh = plsc.VectorSubcoreMesh(core_axis_name="c", subcore_axis_name="s",
                                  num_cores=NC, num_subcores=NS)
    src_r = src.reshape(n_eslices, NS, k * b)
    dst_r = dst.reshape(n_eslices, NS, k, b)
    zeros = jnp.zeros((PAD_N, DW), F32)
    out_ty = [jax.ShapeDtypeStruct((PAD_N, DW), F32)] * n_outs
    body = functools.partial(_scatter_body, n_tables, n_outs, (k, b), tasks)
    kern = pl.kernel(
        body,
        out_type=out_ty,
        mesh=mesh,
        scratch_types=[
            pltpu.VMEM((k * b,), jnp.int32),
            pltpu.VMEM((k, b), jnp.int32),
            pltpu.VMEM((b, DW), F32),
            pltpu.VMEM((b, DW), F32),
            pltpu.VMEM_SHARED((PAD_N, DW), F32),
            pltpu.SemaphoreType.DMA,
        ],
    )
    return kern(src_r, dst_r, zeros, *tables)


def _tc_prep_body(deg3, x, dinv_o, y1_o):
    deg = deg3[0, :, 0:1] + deg3[1, :, 0:1] + 1.0
    dinv = lax.rsqrt(deg)
    dinv_o[...] = dinv
    y = x[...] * dinv[:N_NODES]
    y1_o[...] = jnp.concatenate(
        [y, jnp.zeros((PAD_N - N_NODES, 128), F32)], axis=0)


def _run_prep(deg3, x):
    return pl.pallas_call(
        _tc_prep_body,
        out_shape=[
            jax.ShapeDtypeStruct((PAD_N, 1), F32),
            jax.ShapeDtypeStruct((PAD_N, 128), F32),
        ],
    )(deg3, x)


TN = 640


def _tc_mid_body(dinv, s1a, s1b, y1, w1, b1, w2, *outs):
    di = dinv[...]
    agg = (s1a[...] + s1b[...] + y1[...]) * di
    h = jnp.maximum(jnp.dot(agg, w1[...], preferred_element_type=F32) + b1[...],
                    0.0)
    t = jnp.dot(h, w2[...], preferred_element_type=F32) * di
    t = jnp.concatenate([t, jnp.zeros((t.shape[0], 112), F32)], axis=1)
    for j, o in enumerate(outs):
        o[...] = t[:, 128 * j:128 * (j + 1)]


def _run_mid(dinv, s1a, s1b, y1, w1, b1, w2):
    grid = (PAD_N // TN,)
    node_spec = pl.BlockSpec((TN, 128), lambda i: (i, 0))
    return pl.pallas_call(
        _tc_mid_body,
        grid=grid,
        in_specs=[
            pl.BlockSpec((TN, 1), lambda i: (i, 0)),
            node_spec, node_spec, node_spec,
            pl.BlockSpec((128, 1600), lambda i: (0, 0)),
            pl.BlockSpec((1, 1600), lambda i: (0, 0)),
            pl.BlockSpec((1600, 400), lambda i: (0, 0)),
        ],
        out_specs=[pl.BlockSpec((TN, 128), lambda i: (i, 0))] * 4,
        out_shape=[jax.ShapeDtypeStruct((PAD_N, 128), F32)] * 4,
    )(dinv, s1a, s1b, y1, w1, b1, w2)


def _tc_final_body(*refs):
    dinv, b2 = refs[0], refs[1]
    s2 = refs[2:6]
    y2 = refs[6:10]
    out = refs[10]
    t = jnp.concatenate([s2[j][...] + y2[j][...] for j in range(4)], axis=1)
    out[...] = t[:, :400] * dinv[...] + b2[...]


TNF = 400


def _run_final(dinv, s2, y2, b2):
    grid = (N_NODES // TNF,)
    chunk_spec = pl.BlockSpec((TNF, 128), lambda i: (i, 0))
    return pl.pallas_call(
        _tc_final_body,
        grid=grid,
        in_specs=[pl.BlockSpec((TNF, 1), lambda i: (i, 0)),
                  pl.BlockSpec((1, 400), lambda i: (0, 0))]
                 + [chunk_spec] * 8,
        out_specs=pl.BlockSpec((TNF, 400), lambda i: (i, 0)),
        out_shape=jax.ShapeDtypeStruct((N_NODES, 400), F32),
    )(dinv, b2, *s2, *y2)


@jax.jit
def _kernel_impl(x, edge_index, W1, b1, W2, b2):
    src = edge_index[0]
    dst = edge_index[1]

    deg3 = _run_deg(dst)
    dinv, y1 = _run_prep(deg3, x)
    s1a, s1b = _run_scatter(src, dst, [y1], 2, 2, 125, 40,
                            (((0, 0, 0),), ((0, 1, 1),)))
    y2 = _run_mid(dinv, s1a, s1b, y1, W1, b1.reshape(1, 1600), W2)
    s2 = _run_scatter(src, dst, list(y2), 4, 1, 125, 80,
                      (((0, 0, 0), (2, 2, 0)), ((1, 1, 0), (3, 3, 0))))
    return _run_final(dinv, list(s2), list(y2), b2.reshape(1, 400))


def kernel(x, edge_index, W1, b1, W2, b2):
    return _kernel_impl(x, edge_index, W1, b1, W2, b2)


# speedup vs baseline: 20.5730x; 1.2898x over previous
import functools

import jax
import jax.numpy as jnp
from jax import lax
from jax.experimental import pallas as pl
from jax.experimental.pallas import tpu as pltpu
from jax.experimental.pallas import tpu_sc as plsc

N_NODES = 10000
N_EDGES = 160000
PAD_N = 10240
NC = 2
NS = 16
ROWS_PER_TILE = PAD_N // NS

F32 = jnp.float32


BD = 40
KD = N_EDGES // (NC * NS * BD)


def _deg_body(dst_hbm, ones_hbm, zeros_hbm, out, didx_v, ones_v, acc_sh, dsem):
    c = lax.axis_index("c")
    s = lax.axis_index("s")
    pltpu.sync_copy(zeros_hbm.at[pl.ds(s * ROWS_PER_TILE, ROWS_PER_TILE)],
                    acc_sh.at[pl.ds(s * ROWS_PER_TILE, ROWS_PER_TILE)])
    pltpu.sync_copy(ones_hbm, ones_v)
    pltpu.sync_copy(dst_hbm.at[c].at[s], didx_v)
    plsc.subcore_barrier()

    def step(i, carry):
        j = 5 * i
        cps = [pltpu.async_copy(ones_v, acc_sh.at[didx_v.at[j + u]],
                                dsem, add=True) for u in range(5)]
        for cp in cps:
            cp.wait()
        return carry

    lax.fori_loop(0, KD // 5, step, 0)
    plsc.subcore_barrier()
    pltpu.sync_copy(acc_sh.at[pl.ds(s * ROWS_PER_TILE, ROWS_PER_TILE)],
                    out.at[c].at[pl.ds(s * ROWS_PER_TILE, ROWS_PER_TILE)])


def _run_deg(dst):
    mesh = plsc.VectorSubcoreMesh(core_axis_name="c", subcore_axis_name="s",
                                  num_cores=NC, num_subcores=NS)
    dst_r = dst.reshape(NC, NS, KD, BD)
    ones = jnp.ones((BD, DW), F32)
    zeros = jnp.zeros((PAD_N, DW), F32)
    k = pl.kernel(
        _deg_body,
        out_type=[jax.ShapeDtypeStruct((NC, PAD_N, DW), F32)],
        mesh=mesh,
        scratch_types=[
            pltpu.VMEM((KD, BD), jnp.int32),
            pltpu.VMEM((BD, DW), F32),
            pltpu.VMEM_SHARED((PAD_N, DW), F32),
            pltpu.SemaphoreType.DMA,
        ],
    )
    return k(dst_r, ones, zeros)[0]


DW = 128


CHUNK = 25
NBUF = 4


def _scatter_body(n_tables, n_outs, k_b, tasks, *refs):
    k, b = k_b
    n_outer = k // CHUNK
    (src_hbm, dst_hbm, zeros_hbm), refs = refs[:3], refs[3:]
    tabs, refs = refs[:n_tables], refs[n_tables:]
    outs, refs = refs[:n_outs], refs[n_outs:]
    sidx_v, didx_v = refs[0], refs[1]
    bufs = list(refs[2:2 + NBUF])
    acc_sh, gsem, ssem = refs[2 + NBUF:]
    c = lax.axis_index("c")
    s = lax.axis_index("s")

    for core_id in range(NC):
        for ti, oi, ei in tasks[core_id]:
            @pl.when(c == core_id)
            def _(ti=ti, oi=oi, ei=ei):
                pltpu.sync_copy(
                    zeros_hbm.at[pl.ds(s * ROWS_PER_TILE, ROWS_PER_TILE)],
                    acc_sh.at[pl.ds(s * ROWS_PER_TILE, ROWS_PER_TILE)])
                plsc.subcore_barrier()

                def outer(o, carry):
                    pltpu.sync_copy(src_hbm.at[ei].at[s].at[o], sidx_v)
                    pltpu.sync_copy(dst_hbm.at[ei].at[s].at[o], didx_v)
                    gd = [None] * CHUNK
                    sd = [None] * CHUNK
                    for t in range(CHUNK):
                        bi = t % NBUF
                        if t >= NBUF:
                            sd[t - NBUF].wait()
                        gd[t] = pltpu.async_copy(
                            tabs[ti].at[sidx_v.at[pl.ds(t * b, b)]],
                            bufs[bi], gsem)
                        if t >= 1:
                            gd[t - 1].wait()
                            sd[t - 1] = pltpu.async_copy(
                                bufs[(t - 1) % NBUF],
                                acc_sh.at[didx_v.at[t - 1]], ssem, add=True)
                    gd[CHUNK - 1].wait()
                    sd[CHUNK - 1] = pltpu.async_copy(
                        bufs[(CHUNK - 1) % NBUF],
                        acc_sh.at[didx_v.at[CHUNK - 1]], ssem, add=True)
                    for t in range(CHUNK - NBUF, CHUNK):
                        sd[t].wait()
                    return carry

                lax.fori_loop(0, n_outer, outer, 0)
                plsc.subcore_barrier()
                pltpu.sync_copy(
                    acc_sh.at[pl.ds(s * ROWS_PER_TILE, ROWS_PER_TILE)],
                    outs[oi].at[pl.ds(s * ROWS_PER_TILE, ROWS_PER_TILE)])


def _run_scatter(src, dst, tables, n_outs, n_eslices, k, b, tasks):
    n_tables = len(tables)
    mesh = plsc.VectorSubcoreMesh(core_axis_name="c", subcore_axis_name="s",
                                  num_cores=NC, num_subcores=NS)
    src_r = src.reshape(n_eslices, NS, k // CHUNK, CHUNK * b)
    dst_r = dst.reshape(n_eslices, NS, k // CHUNK, CHUNK, b)
    zeros = jnp.zeros((PAD_N, DW), F32)
    out_ty = [jax.ShapeDtypeStruct((PAD_N, DW), F32)] * n_outs
    body = functools.partial(_scatter_body, n_tables, n_outs, (k, b), tasks)
    kern = pl.kernel(
        body,
        out_type=out_ty,
        mesh=mesh,
        scratch_types=[
            pltpu.VMEM((CHUNK * b,), jnp.int32),
            pltpu.VMEM((CHUNK, b), jnp.int32),
        ] + [pltpu.VMEM((b, DW), F32)] * NBUF + [
            pltpu.VMEM_SHARED((PAD_N, DW), F32),
            pltpu.SemaphoreType.DMA,
            pltpu.SemaphoreType.DMA,
        ],
    )
    return kern(src_r, dst_r, zeros, *tables)


def _tc_prep_body(deg3, x, dinv_o, y1_o):
    deg = deg3[0, :, 0:1] + deg3[1, :, 0:1] + 1.0
    dinv = lax.rsqrt(deg)
    dinv_o[...] = dinv
    y = x[...] * dinv[:N_NODES]
    y1_o[...] = jnp.concatenate(
        [y, jnp.zeros((PAD_N - N_NODES, 128), F32)], axis=0)


def _run_prep(deg3, x):
    return pl.pallas_call(
        _tc_prep_body,
        out_shape=[
            jax.ShapeDtypeStruct((PAD_N, 1), F32),
            jax.ShapeDtypeStruct((PAD_N, 128), F32),
        ],
    )(deg3, x)


TN = 640


def _tc_mid_body(dinv, s1a, s1b, y1, w1, b1, w2, *outs):
    di = dinv[...]
    agg = (s1a[...] + s1b[...] + y1[...]) * di
    h = jnp.maximum(jnp.dot(agg, w1[...], preferred_element_type=F32) + b1[...],
                    0.0)
    t = jnp.dot(h, w2[...], preferred_element_type=F32) * di
    t = jnp.concatenate([t, jnp.zeros((t.shape[0], 112), F32)], axis=1)
    for j, o in enumerate(outs):
        o[...] = t[:, 128 * j:128 * (j + 1)]


def _run_mid(dinv, s1a, s1b, y1, w1, b1, w2):
    grid = (PAD_N // TN,)
    node_spec = pl.BlockSpec((TN, 128), lambda i: (i, 0))
    return pl.pallas_call(
        _tc_mid_body,
        grid=grid,
        in_specs=[
            pl.BlockSpec((TN, 1), lambda i: (i, 0)),
            node_spec, node_spec, node_spec,
            pl.BlockSpec((128, 1600), lambda i: (0, 0)),
            pl.BlockSpec((1, 1600), lambda i: (0, 0)),
            pl.BlockSpec((1600, 400), lambda i: (0, 0)),
        ],
        out_specs=[pl.BlockSpec((TN, 128), lambda i: (i, 0))] * 4,
        out_shape=[jax.ShapeDtypeStruct((PAD_N, 128), F32)] * 4,
    )(dinv, s1a, s1b, y1, w1, b1, w2)


def _tc_final_body(*refs):
    dinv, b2 = refs[0], refs[1]
    s2 = refs[2:6]
    y2 = refs[6:10]
    out = refs[10]
    t = jnp.concatenate([s2[j][...] + y2[j][...] for j in range(4)], axis=1)
    out[...] = t[:, :400] * dinv[...] + b2[...]


TNF = 400


def _run_final(dinv, s2, y2, b2):
    grid = (N_NODES // TNF,)
    chunk_spec = pl.BlockSpec((TNF, 128), lambda i: (i, 0))
    return pl.pallas_call(
        _tc_final_body,
        grid=grid,
        in_specs=[pl.BlockSpec((TNF, 1), lambda i: (i, 0)),
                  pl.BlockSpec((1, 400), lambda i: (0, 0))]
                 + [chunk_spec] * 8,
        out_specs=pl.BlockSpec((TNF, 400), lambda i: (i, 0)),
        out_shape=jax.ShapeDtypeStruct((N_NODES, 400), F32),
    )(dinv, b2, *s2, *y2)


@jax.jit
def _kernel_impl(x, edge_index, W1, b1, W2, b2):
    src = edge_index[0]
    dst = edge_index[1]

    deg3 = _run_deg(dst)
    dinv, y1 = _run_prep(deg3, x)
    s1a, s1b = _run_scatter(src, dst, [y1], 2, 2, 125, 40,
                            (((0, 0, 0),), ((0, 1, 1),)))
    y2 = _run_mid(dinv, s1a, s1b, y1, W1, b1.reshape(1, 1600), W2)
    s2 = _run_scatter(src, dst, list(y2), 4, 1, 125, 80,
                      (((0, 0, 0), (2, 2, 0)), ((1, 1, 0), (3, 3, 0))))
    return _run_final(dinv, list(s2), list(y2), b2.reshape(1, 400))


def kernel(x, edge_index, W1, b1, W2, b2):
    return _kernel_impl(x, edge_index, W1, b1, W2, b2)


# speedup vs baseline: 20.6242x; 1.0025x over previous
import functools

import jax
import jax.numpy as jnp
from jax import lax
from jax.experimental import pallas as pl
from jax.experimental.pallas import tpu as pltpu
from jax.experimental.pallas import tpu_sc as plsc

N_NODES = 10000
N_EDGES = 160000
PAD_N = 10240
NC = 2
NS = 16
ROWS_PER_TILE = PAD_N // NS

F32 = jnp.float32


BD = 40
KD = N_EDGES // (NC * NS * BD)


def _deg_body(dst_hbm, ones_hbm, zeros_hbm, out, didx_v, ones_v, acc_sh, dsem):
    c = lax.axis_index("c")
    s = lax.axis_index("s")
    pltpu.sync_copy(zeros_hbm.at[pl.ds(s * ROWS_PER_TILE, ROWS_PER_TILE)],
                    acc_sh.at[pl.ds(s * ROWS_PER_TILE, ROWS_PER_TILE)])
    pltpu.sync_copy(ones_hbm, ones_v)
    pltpu.sync_copy(dst_hbm.at[c].at[s], didx_v)
    plsc.subcore_barrier()

    def step(i, carry):
        j = 5 * i
        cps = [pltpu.async_copy(ones_v, acc_sh.at[didx_v.at[j + u]],
                                dsem, add=True) for u in range(5)]
        for cp in cps:
            cp.wait()
        return carry

    lax.fori_loop(0, KD // 5, step, 0)
    plsc.subcore_barrier()
    pltpu.sync_copy(acc_sh.at[pl.ds(s * ROWS_PER_TILE, ROWS_PER_TILE)],
                    out.at[c].at[pl.ds(s * ROWS_PER_TILE, ROWS_PER_TILE)])


def _run_deg(dst):
    mesh = plsc.VectorSubcoreMesh(core_axis_name="c", subcore_axis_name="s",
                                  num_cores=NC, num_subcores=NS)
    dst_r = dst.reshape(NC, NS, KD, BD)
    ones = jnp.ones((BD, DW), F32)
    zeros = jnp.zeros((PAD_N, DW), F32)
    k = pl.kernel(
        _deg_body,
        out_type=[jax.ShapeDtypeStruct((NC, PAD_N, DW), F32)],
        mesh=mesh,
        scratch_types=[
            pltpu.VMEM((KD, BD), jnp.int32),
            pltpu.VMEM((BD, DW), F32),
            pltpu.VMEM_SHARED((PAD_N, DW), F32),
            pltpu.SemaphoreType.DMA,
        ],
    )
    return k(dst_r, ones, zeros)[0]


DW = 128


CHUNK = 25
NBUF = 4


def _scatter_body(n_tables, n_outs, k_b, tasks, *refs):
    k, b = k_b
    n_outer = k // CHUNK
    (src_hbm, dst_hbm, zeros_hbm), refs = refs[:3], refs[3:]
    tabs, refs = refs[:n_tables], refs[n_tables:]
    outs, refs = refs[:n_outs], refs[n_outs:]
    sidx_v, didx_v = refs[0], refs[1]
    bufs = list(refs[2:2 + NBUF])
    acc_sh, gsem, ssem = refs[2 + NBUF:]
    c = lax.axis_index("c")
    s = lax.axis_index("s")

    for core_id in range(NC):
        for ti, oi, ei in tasks[core_id]:
            @pl.when(c == core_id)
            def _(ti=ti, oi=oi, ei=ei):
                pltpu.sync_copy(
                    zeros_hbm.at[pl.ds(s * ROWS_PER_TILE, ROWS_PER_TILE)],
                    acc_sh.at[pl.ds(s * ROWS_PER_TILE, ROWS_PER_TILE)])
                plsc.subcore_barrier()

                def outer(o, carry):
                    pltpu.sync_copy(src_hbm.at[ei].at[s].at[o], sidx_v)
                    pltpu.sync_copy(dst_hbm.at[ei].at[s].at[o], didx_v)
                    gd = [None] * CHUNK
                    sd = [None] * CHUNK
                    for t in range(CHUNK):
                        bi = t % NBUF
                        if t >= NBUF:
                            sd[t - NBUF].wait()
                        gd[t] = pltpu.async_copy(
                            tabs[ti].at[sidx_v.at[pl.ds(t * b, b)]],
                            bufs[bi], gsem)
                        if t >= 1:
                            gd[t - 1].wait()
                            sd[t - 1] = pltpu.async_copy(
                                bufs[(t - 1) % NBUF],
                                acc_sh.at[didx_v.at[t - 1]], ssem, add=True)
                    gd[CHUNK - 1].wait()
                    sd[CHUNK - 1] = pltpu.async_copy(
                        bufs[(CHUNK - 1) % NBUF],
                        acc_sh.at[didx_v.at[CHUNK - 1]], ssem, add=True)
                    for t in range(CHUNK - NBUF, CHUNK):
                        sd[t].wait()
                    return carry

                lax.fori_loop(0, n_outer, outer, 0)
                plsc.subcore_barrier()
                pltpu.sync_copy(
                    acc_sh.at[pl.ds(s * ROWS_PER_TILE, ROWS_PER_TILE)],
                    outs[oi].at[pl.ds(s * ROWS_PER_TILE, ROWS_PER_TILE)])


def _run_scatter(src, dst, tables, n_outs, n_eslices, k, b, tasks):
    n_tables = len(tables)
    mesh = plsc.VectorSubcoreMesh(core_axis_name="c", subcore_axis_name="s",
                                  num_cores=NC, num_subcores=NS)
    src_r = src.reshape(n_eslices, NS, k // CHUNK, CHUNK * b)
    dst_r = dst.reshape(n_eslices, NS, k // CHUNK, CHUNK, b)
    zeros = jnp.zeros((PAD_N, DW), F32)
    out_ty = [jax.ShapeDtypeStruct((PAD_N, DW), F32)] * n_outs
    body = functools.partial(_scatter_body, n_tables, n_outs, (k, b), tasks)
    kern = pl.kernel(
        body,
        out_type=out_ty,
        mesh=mesh,
        scratch_types=[
            pltpu.VMEM((CHUNK * b,), jnp.int32),
            pltpu.VMEM((CHUNK, b), jnp.int32),
        ] + [pltpu.VMEM((b, DW), F32)] * NBUF + [
            pltpu.VMEM_SHARED((PAD_N, DW), F32),
            pltpu.SemaphoreType.DMA,
            pltpu.SemaphoreType.DMA,
        ],
    )
    return kern(src_r, dst_r, zeros, *tables)


def _tc_prep_body(deg3, x, dinv_o, y1_o):
    deg = deg3[0, :, 0:1] + deg3[1, :, 0:1] + 1.0
    dinv = lax.rsqrt(deg)
    dinv_o[...] = dinv
    y = x[...] * dinv[:N_NODES]
    y1_o[...] = jnp.concatenate(
        [y, jnp.zeros((PAD_N - N_NODES, 128), F32)], axis=0)


def _run_prep(deg3, x):
    return pl.pallas_call(
        _tc_prep_body,
        out_shape=[
            jax.ShapeDtypeStruct((PAD_N, 1), F32),
            jax.ShapeDtypeStruct((PAD_N, 128), F32),
        ],
    )(deg3, x)


TN = 1024


def _tc_mid_body(dinv, s1a, s1b, y1, w1, b1, w2, *outs):
    di = dinv[...]
    agg = (s1a[...] + s1b[...] + y1[...]) * di
    h = jnp.maximum(
        jnp.dot(agg.astype(jnp.bfloat16), w1[...].astype(jnp.bfloat16),
                preferred_element_type=F32) + b1[...], 0.0)
    t = jnp.dot(h.astype(jnp.bfloat16), w2[...].astype(jnp.bfloat16),
                preferred_element_type=F32) * di
    t = jnp.concatenate([t, jnp.zeros((t.shape[0], 112), F32)], axis=1)
    for j, o in enumerate(outs):
        o[...] = t[:, 128 * j:128 * (j + 1)]


def _run_mid(dinv, s1a, s1b, y1, w1, b1, w2):
    grid = (PAD_N // TN,)
    node_spec = pl.BlockSpec((TN, 128), lambda i: (i, 0))
    return pl.pallas_call(
        _tc_mid_body,
        grid=grid,
        in_specs=[
            pl.BlockSpec((TN, 1), lambda i: (i, 0)),
            node_spec, node_spec, node_spec,
            pl.BlockSpec((128, 1600), lambda i: (0, 0)),
            pl.BlockSpec((1, 1600), lambda i: (0, 0)),
            pl.BlockSpec((1600, 400), lambda i: (0, 0)),
        ],
        out_specs=[pl.BlockSpec((TN, 128), lambda i: (i, 0))] * 4,
        out_shape=[jax.ShapeDtypeStruct((PAD_N, 128), F32)] * 4,
    )(dinv, s1a, s1b, y1, w1, b1, w2)


def _tc_final_body(*refs):
    dinv, b2 = refs[0], refs[1]
    s2 = refs[2:6]
    y2 = refs[6:10]
    out = refs[10]
    t = jnp.concatenate([s2[j][...] + y2[j][...] for j in range(4)], axis=1)
    out[...] = t[:, :400] * dinv[...] + b2[...]


TNF = 400


def _run_final(dinv, s2, y2, b2):
    grid = (N_NODES // TNF,)
    chunk_spec = pl.BlockSpec((TNF, 128), lambda i: (i, 0))
    return pl.pallas_call(
        _tc_final_body,
        grid=grid,
        in_specs=[pl.BlockSpec((TNF, 1), lambda i: (i, 0)),
                  pl.BlockSpec((1, 400), lambda i: (0, 0))]
                 + [chunk_spec] * 8,
        out_specs=pl.BlockSpec((TNF, 400), lambda i: (i, 0)),
        out_shape=jax.ShapeDtypeStruct((N_NODES, 400), F32),
    )(dinv, b2, *s2, *y2)


@jax.jit
def _kernel_impl(x, edge_index, W1, b1, W2, b2):
    src = edge_index[0]
    dst = edge_index[1]

    deg3 = _run_deg(dst)
    dinv, y1 = _run_prep(deg3, x)
    s1a, s1b = _run_scatter(src, dst, [y1], 2, 2, 125, 40,
                            (((0, 0, 0),), ((0, 1, 1),)))
    y2 = _run_mid(dinv, s1a, s1b, y1, W1, b1.reshape(1, 1600), W2)
    s2 = _run_scatter(src, dst, list(y2), 4, 1, 125, 80,
                      (((0, 0, 0), (2, 2, 0)), ((1, 1, 0), (3, 3, 0))))
    return _run_final(dinv, list(s2), list(y2), b2.reshape(1, 400))


def kernel(x, edge_index, W1, b1, W2, b2):
    return _kernel_impl(x, edge_index, W1, b1, W2, b2)


# speedup vs baseline: 20.6692x; 1.0022x over previous
import functools

import jax
import jax.numpy as jnp
from jax import lax
from jax.experimental import pallas as pl
from jax.experimental.pallas import tpu as pltpu
from jax.experimental.pallas import tpu_sc as plsc

N_NODES = 10000
N_EDGES = 160000
PAD_N = 10240
NC = 2
NS = 16
ROWS_PER_TILE = PAD_N // NS

F32 = jnp.float32


BD = 100
KD = N_EDGES // (NC * NS * BD)


def _deg_body(dst_hbm, ones_hbm, zeros_hbm, out, didx_v, ones_v, acc_sh, dsem):
    c = lax.axis_index("c")
    s = lax.axis_index("s")
    pltpu.sync_copy(zeros_hbm.at[pl.ds(s * ROWS_PER_TILE, ROWS_PER_TILE)],
                    acc_sh.at[pl.ds(s * ROWS_PER_TILE, ROWS_PER_TILE)])
    pltpu.sync_copy(ones_hbm, ones_v)
    pltpu.sync_copy(dst_hbm.at[c].at[s], didx_v)
    plsc.subcore_barrier()

    def step(i, carry):
        j = 5 * i
        cps = [pltpu.async_copy(ones_v, acc_sh.at[didx_v.at[j + u]],
                                dsem, add=True) for u in range(5)]
        for cp in cps:
            cp.wait()
        return carry

    lax.fori_loop(0, KD // 5, step, 0)
    plsc.subcore_barrier()
    pltpu.sync_copy(acc_sh.at[pl.ds(s * ROWS_PER_TILE, ROWS_PER_TILE)],
                    out.at[c].at[pl.ds(s * ROWS_PER_TILE, ROWS_PER_TILE)])


def _run_deg(dst):
    mesh = plsc.VectorSubcoreMesh(core_axis_name="c", subcore_axis_name="s",
                                  num_cores=NC, num_subcores=NS)
    dst_r = dst.reshape(NC, NS, KD, BD)
    ones = jnp.ones((BD, DW), F32)
    zeros = jnp.zeros((PAD_N, DW), F32)
    k = pl.kernel(
        _deg_body,
        out_type=[jax.ShapeDtypeStruct((NC, PAD_N, DW), F32)],
        mesh=mesh,
        scratch_types=[
            pltpu.VMEM((KD, BD), jnp.int32),
            pltpu.VMEM((BD, DW), F32),
            pltpu.VMEM_SHARED((PAD_N, DW), F32),
            pltpu.SemaphoreType.DMA,
        ],
    )
    return k(dst_r, ones, zeros)[0]


DW = 128


CHUNK = 25
NBUF = 4


def _scatter_body(n_tables, n_outs, k_b, tasks, *refs):
    k, b = k_b
    n_outer = k // CHUNK
    (src_hbm, dst_hbm, zeros_hbm), refs = refs[:3], refs[3:]
    tabs, refs = refs[:n_tables], refs[n_tables:]
    outs, refs = refs[:n_outs], refs[n_outs:]
    sidx_v, didx_v = refs[0], refs[1]
    bufs = list(refs[2:2 + NBUF])
    acc_sh, gsem, ssem = refs[2 + NBUF:]
    c = lax.axis_index("c")
    s = lax.axis_index("s")

    for core_id in range(NC):
        for ti, oi, ei in tasks[core_id]:
            @pl.when(c == core_id)
            def _(ti=ti, oi=oi, ei=ei):
                pltpu.sync_copy(
                    zeros_hbm.at[pl.ds(s * ROWS_PER_TILE, ROWS_PER_TILE)],
                    acc_sh.at[pl.ds(s * ROWS_PER_TILE, ROWS_PER_TILE)])
                plsc.subcore_barrier()

                def outer(o, carry):
                    pltpu.sync_copy(src_hbm.at[ei].at[s].at[o], sidx_v)
                    pltpu.sync_copy(dst_hbm.at[ei].at[s].at[o], didx_v)
                    gd = [None] * CHUNK
                    sd = [None] * CHUNK
                    for t in range(CHUNK):
                        bi = t % NBUF
                        if t >= NBUF:
                            sd[t - NBUF].wait()
                        gd[t] = pltpu.async_copy(
                            tabs[ti].at[sidx_v.at[pl.ds(t * b, b)]],
                            bufs[bi], gsem)
                        if t >= 1:
                            gd[t - 1].wait()
                            sd[t - 1] = pltpu.async_copy(
                                bufs[(t - 1) % NBUF],
                                acc_sh.at[didx_v.at[t - 1]], ssem, add=True)
                    gd[CHUNK - 1].wait()
                    sd[CHUNK - 1] = pltpu.async_copy(
                        bufs[(CHUNK - 1) % NBUF],
                        acc_sh.at[didx_v.at[CHUNK - 1]], ssem, add=True)
                    for t in range(CHUNK - NBUF, CHUNK):
                        sd[t].wait()
                    return carry

                lax.fori_loop(0, n_outer, outer, 0)
                plsc.subcore_barrier()
                pltpu.sync_copy(
                    acc_sh.at[pl.ds(s * ROWS_PER_TILE, ROWS_PER_TILE)],
                    outs[oi].at[pl.ds(s * ROWS_PER_TILE, ROWS_PER_TILE)])


def _run_scatter(src, dst, tables, n_outs, n_eslices, k, b, tasks):
    n_tables = len(tables)
    mesh = plsc.VectorSubcoreMesh(core_axis_name="c", subcore_axis_name="s",
                                  num_cores=NC, num_subcores=NS)
    src_r = src.reshape(n_eslices, NS, k // CHUNK, CHUNK * b)
    dst_r = dst.reshape(n_eslices, NS, k // CHUNK, CHUNK, b)
    zeros = jnp.zeros((PAD_N, DW), F32)
    out_ty = [jax.ShapeDtypeStruct((PAD_N, DW), F32)] * n_outs
    body = functools.partial(_scatter_body, n_tables, n_outs, (k, b), tasks)
    kern = pl.kernel(
        body,
        out_type=out_ty,
        mesh=mesh,
        scratch_types=[
            pltpu.VMEM((CHUNK * b,), jnp.int32),
            pltpu.VMEM((CHUNK, b), jnp.int32),
        ] + [pltpu.VMEM((b, DW), F32)] * NBUF + [
            pltpu.VMEM_SHARED((PAD_N, DW), F32),
            pltpu.SemaphoreType.DMA,
            pltpu.SemaphoreType.DMA,
        ],
    )
    return kern(src_r, dst_r, zeros, *tables)


def _tc_prep_body(deg3, x, dinv_o, y1_o):
    deg = deg3[0, :, 0:1] + deg3[1, :, 0:1] + 1.0
    dinv = lax.rsqrt(deg)
    dinv_o[...] = dinv
    y = x[...] * dinv[:N_NODES]
    y1_o[...] = jnp.concatenate(
        [y, jnp.zeros((PAD_N - N_NODES, 128), F32)], axis=0)


def _run_prep(deg3, x):
    return pl.pallas_call(
        _tc_prep_body,
        out_shape=[
            jax.ShapeDtypeStruct((PAD_N, 1), F32),
            jax.ShapeDtypeStruct((PAD_N, 128), F32),
        ],
    )(deg3, x)


TN = 1024


def _tc_mid_body(dinv, s1a, s1b, y1, w1, b1, w2, *outs):
    di = dinv[...]
    agg = (s1a[...] + s1b[...] + y1[...]) * di
    h = jnp.maximum(
        jnp.dot(agg.astype(jnp.bfloat16), w1[...].astype(jnp.bfloat16),
                preferred_element_type=F32) + b1[...], 0.0)
    t = jnp.dot(h.astype(jnp.bfloat16), w2[...].astype(jnp.bfloat16),
                preferred_element_type=F32) * di
    t = jnp.concatenate([t, jnp.zeros((t.shape[0], 112), F32)], axis=1)
    for j, o in enumerate(outs):
        o[...] = t[:, 128 * j:128 * (j + 1)]


def _run_mid(dinv, s1a, s1b, y1, w1, b1, w2):
    grid = (PAD_N // TN,)
    node_spec = pl.BlockSpec((TN, 128), lambda i: (i, 0))
    return pl.pallas_call(
        _tc_mid_body,
        grid=grid,
        in_specs=[
            pl.BlockSpec((TN, 1), lambda i: (i, 0)),
            node_spec, node_spec, node_spec,
            pl.BlockSpec((128, 1600), lambda i: (0, 0)),
            pl.BlockSpec((1, 1600), lambda i: (0, 0)),
            pl.BlockSpec((1600, 400), lambda i: (0, 0)),
        ],
        out_specs=[pl.BlockSpec((TN, 128), lambda i: (i, 0))] * 4,
        out_shape=[jax.ShapeDtypeStruct((PAD_N, 128), F32)] * 4,
    )(dinv, s1a, s1b, y1, w1, b1, w2)


def _tc_final_body(*refs):
    dinv, b2 = refs[0], refs[1]
    s2 = refs[2:6]
    y2 = refs[6:10]
    out = refs[10]
    t = jnp.concatenate([s2[j][...] + y2[j][...] for j in range(4)], axis=1)
    out[...] = t[:, :400] * dinv[...] + b2[...]


TNF = 400


def _run_final(dinv, s2, y2, b2):
    grid = (N_NODES // TNF,)
    chunk_spec = pl.BlockSpec((TNF, 128), lambda i: (i, 0))
    return pl.pallas_call(
        _tc_final_body,
        grid=grid,
        in_specs=[pl.BlockSpec((TNF, 1), lambda i: (i, 0)),
                  pl.BlockSpec((1, 400), lambda i: (0, 0))]
                 + [chunk_spec] * 8,
        out_specs=pl.BlockSpec((TNF, 400), lambda i: (i, 0)),
        out_shape=jax.ShapeDtypeStruct((N_NODES, 400), F32),
    )(dinv, b2, *s2, *y2)


@jax.jit
def _kernel_impl(x, edge_index, W1, b1, W2, b2):
    src = edge_index[0]
    dst = edge_index[1]

    deg3 = _run_deg(dst)
    dinv, y1 = _run_prep(deg3, x)
    s1a, s1b = _run_scatter(src, dst, [y1], 2, 2, 125, 40,
                            (((0, 0, 0),), ((0, 1, 1),)))
    y2 = _run_mid(dinv, s1a, s1b, y1, W1, b1.reshape(1, 1600), W2)
    s2 = _run_scatter(src, dst, list(y2), 4, 1, 125, 80,
                      (((0, 0, 0), (2, 2, 0)), ((1, 1, 0), (3, 3, 0))))
    return _run_final(dinv, list(s2), list(y2), b2.reshape(1, 400))


def kernel(x, edge_index, W1, b1, W2, b2):
    return _kernel_impl(x, edge_index, W1, b1, W2, b2)
